# Initial kernel scaffold; baseline (speedup 1.0000x reference)
import jax
import jax.numpy as jnp
from jax.experimental import pallas as pl


def kernel(node_embedding, label_ids, segment_ids, label_table, W, b):
    raise NotImplementedError("write your pallas kernel here")


# speedup vs baseline: 3.2605x; 3.2605x over previous
import jax
import jax.numpy as jnp
from jax.experimental import pallas as pl
from jax.experimental.pallas import tpu as pltpu

_TOTAL = 16384
_B = 16
_D_TXT = 1024
_D_LBL = 512
_N_LABELS = 64
_TILE = 256
_GRID = _TOTAL // _TILE


def _body(x_ref, seg_ref, lbl_ref, lt_ref, w_ref, b_ref, out_ref,
          acc_ref, m_ref, d_ref):
    i = pl.program_id(0)

    @pl.when(i == 0)
    def _init():
        acc_ref[...] = jnp.zeros_like(acc_ref)
        m_ref[...] = jnp.full_like(m_ref, -jnp.inf)
        d_ref[...] = jnp.zeros_like(d_ref)

    x = x_ref[...]
    seg = seg_ref[...]
    lbl = lbl_ref[...]

    w_all = w_ref[...]
    w_lbl = w_all[0:_D_LBL, :]
    w_txt = w_all[_D_LBL:_D_LBL + _D_TXT, :]

    lbl_scores = jnp.dot(lt_ref[...], w_lbl,
                         preferred_element_type=jnp.float32)
    iota_lbl = jax.lax.broadcasted_iota(jnp.int32, (_TILE, _N_LABELS), 1)
    lf = (lbl == iota_lbl).astype(jnp.float32)
    s_lbl = jnp.dot(lf, lbl_scores,
                    preferred_element_type=jnp.float32)

    s_txt = jnp.dot(x, w_txt, preferred_element_type=jnp.float32)
    s = s_txt + s_lbl + b_ref[0, 0]

    iota_seg = jax.lax.broadcasted_iota(jnp.int32, (_TILE, _B), 1)
    onehot = seg == iota_seg
    of = onehot.astype(jnp.float32)

    tile_max = jnp.max(jnp.where(onehot, s, -jnp.inf),
                       axis=0, keepdims=True)
    m_old = m_ref[...]
    m_new = jnp.maximum(m_old, tile_max)
    rescale = jnp.where(m_old == -jnp.inf, 0.0, jnp.exp(m_old - m_new))
    m_ref[...] = m_new

    m_node = jnp.sum(of * m_new, axis=1, keepdims=True)
    e = jnp.exp(s - m_node)
    oe = of * e

    d_ref[...] = d_ref[...] * rescale + jnp.sum(oe, axis=0, keepdims=True)
    contrib = jax.lax.dot_general(
        x, oe, dimension_numbers=(((0,), (0,)), ((), ())),
        preferred_element_type=jnp.float32)
    acc_ref[...] = acc_ref[...] * rescale + contrib

    @pl.when(i == _GRID - 1)
    def _finish():
        out_ref[...] = acc_ref[...] / (d_ref[...] + 1e-9)


def kernel(node_embedding, label_ids, segment_ids, label_table, W, b):
    seg3 = segment_ids.astype(jnp.int32).reshape(_GRID, _TILE, 1)
    lbl3 = label_ids.astype(jnp.int32).reshape(_GRID, _TILE, 1)
    b2 = b.reshape(1, 1)

    out = pl.pallas_call(
        _body,
        grid=(_GRID,),
        in_specs=[
            pl.BlockSpec((_TILE, _D_TXT), lambda i: (i, 0)),
            pl.BlockSpec((None, _TILE, 1), lambda i: (i, 0, 0)),
            pl.BlockSpec((None, _TILE, 1), lambda i: (i, 0, 0)),
            pl.BlockSpec((_N_LABELS, _D_LBL), lambda i: (0, 0)),
            pl.BlockSpec((_D_LBL + _D_TXT, 1), lambda i: (0, 0)),
            pl.BlockSpec((1, 1), lambda i: (0, 0)),
        ],
        out_specs=pl.BlockSpec((_D_TXT, _B), lambda i: (0, 0)),
        out_shape=jax.ShapeDtypeStruct((_D_TXT, _B), jnp.float32),
        scratch_shapes=[
            pltpu.VMEM((_D_TXT, _B), jnp.float32),
            pltpu.VMEM((1, _B), jnp.float32),
            pltpu.VMEM((1, _B), jnp.float32),
        ],
    )(node_embedding, seg3, lbl3, label_table, W, b2)
    return out.T


# speedup vs baseline: 5.3119x; 1.6291x over previous
import jax
import jax.numpy as jnp
from jax.experimental import pallas as pl
from jax.experimental.pallas import tpu as pltpu

_TOTAL = 16384
_B = 16
_D_TXT = 1024
_D_LBL = 512
_N_LABELS = 64
_TILE = 1024
_GRID = _TOTAL // _TILE


def _body(x_ref, seg_ref, lbl_ref, lt_ref, w_ref, b_ref, out_ref,
          acc_ref, m_ref, d_ref):
    i = pl.program_id(0)

    @pl.when(i == 0)
    def _init():
        acc_ref[...] = jnp.zeros_like(acc_ref)
        m_ref[...] = jnp.full_like(m_ref, -jnp.inf)
        d_ref[...] = jnp.zeros_like(d_ref)

    x = x_ref[...]
    seg = seg_ref[...]
    lbl = lbl_ref[...]

    w_all = w_ref[...]
    w_lbl = w_all[0:_D_LBL, :]
    w_txt = w_all[_D_LBL:_D_LBL + _D_TXT, :]

    lbl_scores = jnp.dot(lt_ref[...], w_lbl,
                         preferred_element_type=jnp.float32)
    iota_lbl = jax.lax.broadcasted_iota(jnp.int32, (_TILE, _N_LABELS), 1)
    lf = (lbl == iota_lbl).astype(jnp.float32)
    s_lbl = jnp.dot(lf, lbl_scores,
                    preferred_element_type=jnp.float32)

    s_txt = jnp.dot(x, w_txt, preferred_element_type=jnp.float32)
    s = s_txt + s_lbl + b_ref[0, 0]

    iota_seg = jax.lax.broadcasted_iota(jnp.int32, (_TILE, _B), 1)
    onehot = seg == iota_seg
    of = onehot.astype(jnp.float32)

    tile_max = jnp.max(jnp.where(onehot, s, -jnp.inf),
                       axis=0, keepdims=True)
    m_old = m_ref[...]
    m_new = jnp.maximum(m_old, tile_max)
    rescale = jnp.where(m_old == -jnp.inf, 0.0, jnp.exp(m_old - m_new))
    m_ref[...] = m_new

    m_node = jnp.sum(of * m_new, axis=1, keepdims=True)
    e = jnp.exp(s - m_node)
    oe = of * e

    d_ref[...] = d_ref[...] * rescale + jnp.sum(oe, axis=0, keepdims=True)
    contrib = jax.lax.dot_general(
        x, oe, dimension_numbers=(((0,), (0,)), ((), ())),
        preferred_element_type=jnp.float32)
    acc_ref[...] = acc_ref[...] * rescale + contrib

    @pl.when(i == _GRID - 1)
    def _finish():
        out_ref[...] = acc_ref[...] / (d_ref[...] + 1e-9)


def kernel(node_embedding, label_ids, segment_ids, label_table, W, b):
    seg3 = segment_ids.astype(jnp.int32).reshape(_GRID, _TILE, 1)
    lbl3 = label_ids.astype(jnp.int32).reshape(_GRID, _TILE, 1)
    b2 = b.reshape(1, 1)

    out = pl.pallas_call(
        _body,
        grid=(_GRID,),
        in_specs=[
            pl.BlockSpec((_TILE, _D_TXT), lambda i: (i, 0)),
            pl.BlockSpec((None, _TILE, 1), lambda i: (i, 0, 0)),
            pl.BlockSpec((None, _TILE, 1), lambda i: (i, 0, 0)),
            pl.BlockSpec((_N_LABELS, _D_LBL), lambda i: (0, 0)),
            pl.BlockSpec((_D_LBL + _D_TXT, 1), lambda i: (0, 0)),
            pl.BlockSpec((1, 1), lambda i: (0, 0)),
        ],
        out_specs=pl.BlockSpec((_D_TXT, _B), lambda i: (0, 0)),
        out_shape=jax.ShapeDtypeStruct((_D_TXT, _B), jnp.float32),
        scratch_shapes=[
            pltpu.VMEM((_D_TXT, _B), jnp.float32),
            pltpu.VMEM((1, _B), jnp.float32),
            pltpu.VMEM((1, _B), jnp.float32),
        ],
    )(node_embedding, seg3, lbl3, label_table, W, b2)
    return out.T


# speedup vs baseline: 5.6581x; 1.0652x over previous
import jax
import jax.numpy as jnp
from jax.experimental import pallas as pl
from jax.experimental.pallas import tpu as pltpu

_TOTAL = 16384
_B = 16
_D_TXT = 1024
_D_LBL = 512
_N_LABELS = 64
_TILE = 2048
_GRID = _TOTAL // _TILE


def _body(x_ref, seg_ref, lbl_ref, lt_ref, w_ref, b_ref, out_ref,
          acc_ref, m_ref, d_ref):
    i = pl.program_id(0)

    @pl.when(i == 0)
    def _init():
        acc_ref[...] = jnp.zeros_like(acc_ref)
        m_ref[...] = jnp.full_like(m_ref, -jnp.inf)
        d_ref[...] = jnp.zeros_like(d_ref)

    x = x_ref[...]
    seg = seg_ref[...]
    lbl = lbl_ref[...]

    w_all = w_ref[...]
    w_lbl = w_all[0:_D_LBL, :]
    w_txt = w_all[_D_LBL:_D_LBL + _D_TXT, :]

    lbl_scores = jnp.dot(lt_ref[...], w_lbl,
                         preferred_element_type=jnp.float32)
    iota_lbl = jax.lax.broadcasted_iota(jnp.int32, (_TILE, _N_LABELS), 1)
    lf = (lbl == iota_lbl).astype(jnp.float32)
    s_lbl = jnp.dot(lf, lbl_scores,
                    preferred_element_type=jnp.float32)

    s_txt = jnp.dot(x, w_txt, preferred_element_type=jnp.float32)
    s = s_txt + s_lbl + b_ref[0, 0]

    iota_seg = jax.lax.broadcasted_iota(jnp.int32, (_TILE, _B), 1)
    onehot = seg == iota_seg
    of = onehot.astype(jnp.float32)

    tile_max = jnp.max(jnp.where(onehot, s, -jnp.inf),
                       axis=0, keepdims=True)
    m_old = m_ref[...]
    m_new = jnp.maximum(m_old, tile_max)
    rescale = jnp.where(m_old == -jnp.inf, 0.0, jnp.exp(m_old - m_new))
    m_ref[...] = m_new

    m_node = jnp.sum(of * m_new, axis=1, keepdims=True)
    e = jnp.exp(s - m_node)
    oe = of * e

    d_ref[...] = d_ref[...] * rescale + jnp.sum(oe, axis=0, keepdims=True)
    contrib = jax.lax.dot_general(
        x, oe, dimension_numbers=(((0,), (0,)), ((), ())),
        preferred_element_type=jnp.float32)
    acc_ref[...] = acc_ref[...] * rescale + contrib

    @pl.when(i == _GRID - 1)
    def _finish():
        out_ref[...] = acc_ref[...] / (d_ref[...] + 1e-9)


def kernel(node_embedding, label_ids, segment_ids, label_table, W, b):
    seg3 = segment_ids.astype(jnp.int32).reshape(_GRID, _TILE, 1)
    lbl3 = label_ids.astype(jnp.int32).reshape(_GRID, _TILE, 1)
    b2 = b.reshape(1, 1)

    out = pl.pallas_call(
        _body,
        grid=(_GRID,),
        in_specs=[
            pl.BlockSpec((_TILE, _D_TXT), lambda i: (i, 0)),
            pl.BlockSpec((None, _TILE, 1), lambda i: (i, 0, 0)),
            pl.BlockSpec((None, _TILE, 1), lambda i: (i, 0, 0)),
            pl.BlockSpec((_N_LABELS, _D_LBL), lambda i: (0, 0)),
            pl.BlockSpec((_D_LBL + _D_TXT, 1), lambda i: (0, 0)),
            pl.BlockSpec((1, 1), lambda i: (0, 0)),
        ],
        out_specs=pl.BlockSpec((_D_TXT, _B), lambda i: (0, 0)),
        out_shape=jax.ShapeDtypeStruct((_D_TXT, _B), jnp.float32),
        scratch_shapes=[
            pltpu.VMEM((_D_TXT, _B), jnp.float32),
            pltpu.VMEM((1, _B), jnp.float32),
            pltpu.VMEM((1, _B), jnp.float32),
        ],
    )(node_embedding, seg3, lbl3, label_table, W, b2)
    return out.T
